# baseline (speedup 1.0000x reference)
import jax
import jax.numpy as jnp
from jax.experimental import pallas as pl
from jax.experimental.pallas import tpu as pltpu

_CLS_PAD = 128
_BM = 2000
_NBUF = 3
_WSPLIT = 2


def _mm_kernel(x_ref, w_ref, bc_ref, bb_ref, sc_hbm, pd_hbm,
               sc_buf, pd_buf, sc_sem, pd_sem):
    nsteps = sc_hbm.shape[0] // _BM
    kc = sc_hbm.shape[1]
    i = pl.program_id(0)
    s = jax.lax.rem(i, _NBUF)
    rows = _BM // _WSPLIT

    def sc_copy(c, slot, part):
        return pltpu.make_async_copy(
            sc_buf.at[slot, pl.ds(part * rows, rows), :],
            sc_hbm.at[pl.ds(c * _BM + part * rows, rows), :],
            sc_sem.at[slot, part])

    def pd_copy(c, slot, part):
        return pltpu.make_async_copy(
            pd_buf.at[slot, pl.ds(part * rows, rows), :],
            pd_hbm.at[pl.ds(c * _BM + part * rows, rows), :],
            pd_sem.at[slot, part])

    @pl.when(i >= _NBUF)
    def _drain():
        for p in range(_WSPLIT):
            sc_copy(i - _NBUF, s, p).wait()
            pd_copy(i - _NBUF, s, p).wait()

    y = jnp.dot(x_ref[...].astype(jnp.bfloat16), w_ref[...],
                preferred_element_type=jnp.float32)
    sc_buf[s] = y[:, :kc] + bc_ref[...]
    pd_buf[s] = y[:, _CLS_PAD:_CLS_PAD + pd_buf.shape[2]] + bb_ref[...]
    for p in range(_WSPLIT):
        sc_copy(i, s, p).start()
        pd_copy(i, s, p).start()

    @pl.when(i == nsteps - 1)
    def _epilogue():
        for k in range(_NBUF):
            c = nsteps - _NBUF + k
            slot = jax.lax.rem(jnp.int32(c), _NBUF)
            for p in range(_WSPLIT):
                sc_copy(c, slot, p).wait()
                pd_copy(c, slot, p).wait()


def kernel(x, W_cls, b_cls, W_box, b_box):
    if x.ndim > 2:
        x = x.reshape(x.shape[0], -1)
    n, d = x.shape
    kc = W_cls.shape[1]
    kb = W_box.shape[1]
    assert n % _BM == 0 and n // _BM >= _NBUF and kc <= _CLS_PAD

    w_cat = jnp.concatenate(
        [jnp.pad(W_cls, ((0, 0), (0, _CLS_PAD - kc))), W_box],
        axis=1).astype(jnp.bfloat16)
    bc2 = b_cls.reshape(1, kc)
    bb2 = b_box.reshape(1, kb)

    scores, deltas = pl.pallas_call(
        _mm_kernel,
        grid=(n // _BM,),
        in_specs=[
            pl.BlockSpec((_BM, d), lambda i: (i, 0)),
            pl.BlockSpec((d, _CLS_PAD + kb), lambda i: (0, 0)),
            pl.BlockSpec((1, kc), lambda i: (0, 0)),
            pl.BlockSpec((1, kb), lambda i: (0, 0)),
        ],
        out_specs=[
            pl.BlockSpec(memory_space=pl.ANY),
            pl.BlockSpec(memory_space=pl.ANY),
        ],
        out_shape=[
            jax.ShapeDtypeStruct((n, kc), jnp.float32),
            jax.ShapeDtypeStruct((n, kb), jnp.float32),
        ],
        scratch_shapes=[
            pltpu.VMEM((_NBUF, _BM, kc), jnp.float32),
            pltpu.VMEM((_NBUF, _BM, kb), jnp.float32),
            pltpu.SemaphoreType.DMA((_NBUF, _WSPLIT)),
            pltpu.SemaphoreType.DMA((_NBUF, _WSPLIT)),
        ],
        compiler_params=pltpu.CompilerParams(
            dimension_semantics=("arbitrary",),
        ),
    )(x, w_cat, bc2, bb2)
    return (scores, deltas)
